# Initial kernel scaffold; baseline (speedup 1.0000x reference)
import jax
import jax.numpy as jnp
from jax.experimental import pallas as pl


def kernel(x, edge_index, edge_weight, W_xz, b_xz, W_hz, b_hz, W_xr, b_xr, W_hr, b_hr, W_xh, b_xh, W_hh, b_hh, W_lin, b_lin):
    raise NotImplementedError("write your pallas kernel here")


# speedup vs baseline: 1.7908x; 1.7908x over previous
import jax
import jax.numpy as jnp
from jax.experimental import pallas as pl

_N = 10000
_C = 128
_BLK = 1000


def _body(x_ref, wcat_ref, bcat_ref, wlin_ref, blin_ref, o_ref):
    t = jnp.dot(x_ref[...], wcat_ref[...], preferred_element_type=jnp.float32)
    t = t + bcat_ref[...]
    a = t[:, :_C]
    b = t[:, _C:]
    hpre = jax.nn.sigmoid(-a) * jnp.tanh(b)
    h = jnp.where(hpre > 0, hpre, jnp.exp(hpre) - 1.0)
    o_ref[...] = (
        jnp.dot(h, wlin_ref[...], preferred_element_type=jnp.float32)
        + blin_ref[...]
    )


def kernel(x, edge_index, edge_weight, W_xz, b_xz, W_hz, b_hz, W_xr, b_xr,
           W_hr, b_hr, W_xh, b_xh, W_hh, b_hh, W_lin, b_lin):
    wcat = jnp.concatenate([W_xz, W_xh], axis=1)
    bcat = jnp.concatenate([b_xz + b_hz, b_xh + b_hh]).reshape(1, 2 * _C)
    blin = b_lin.reshape(1, _C)

    grid = (_N // _BLK,)
    return pl.pallas_call(
        _body,
        grid=grid,
        in_specs=[
            pl.BlockSpec((_BLK, _C), lambda i: (i, 0)),
            pl.BlockSpec((_C, 2 * _C), lambda i: (0, 0)),
            pl.BlockSpec((1, 2 * _C), lambda i: (0, 0)),
            pl.BlockSpec((_C, _C), lambda i: (0, 0)),
            pl.BlockSpec((1, _C), lambda i: (0, 0)),
        ],
        out_specs=pl.BlockSpec((_BLK, _C), lambda i: (i, 0)),
        out_shape=jax.ShapeDtypeStruct((_N, _C), jnp.float32),
    )(x, wcat, bcat, W_lin, blin)


# speedup vs baseline: 2.1567x; 1.2044x over previous
import jax
import jax.numpy as jnp
from jax.experimental import pallas as pl

_N = 10000
_C = 128
_BLK = 2000


def _body(x_ref, wcat_ref, bcat_ref, wlin_ref, blin_ref, o_ref):
    t = jnp.dot(x_ref[...], wcat_ref[...], preferred_element_type=jnp.float32)
    t = t + bcat_ref[...]
    a = t[:, :_C]
    b = t[:, _C:]
    hpre = jax.nn.sigmoid(-a) * jnp.tanh(b)
    h = jnp.where(hpre > 0, hpre, jnp.exp(hpre) - 1.0)
    o_ref[...] = (
        jnp.dot(h, wlin_ref[...], preferred_element_type=jnp.float32)
        + blin_ref[...]
    )


def kernel(x, edge_index, edge_weight, W_xz, b_xz, W_hz, b_hz, W_xr, b_xr,
           W_hr, b_hr, W_xh, b_xh, W_hh, b_hh, W_lin, b_lin):
    wcat = jnp.concatenate([W_xz, W_xh], axis=1)
    bcat = jnp.concatenate([b_xz + b_hz, b_xh + b_hh]).reshape(1, 2 * _C)
    blin = b_lin.reshape(1, _C)

    grid = (_N // _BLK,)
    return pl.pallas_call(
        _body,
        grid=grid,
        in_specs=[
            pl.BlockSpec((_BLK, _C), lambda i: (i, 0)),
            pl.BlockSpec((_C, 2 * _C), lambda i: (0, 0)),
            pl.BlockSpec((1, 2 * _C), lambda i: (0, 0)),
            pl.BlockSpec((_C, _C), lambda i: (0, 0)),
            pl.BlockSpec((1, _C), lambda i: (0, 0)),
        ],
        out_specs=pl.BlockSpec((_BLK, _C), lambda i: (i, 0)),
        out_shape=jax.ShapeDtypeStruct((_N, _C), jnp.float32),
    )(x, wcat, bcat, W_lin, blin)


# speedup vs baseline: 2.5330x; 1.1745x over previous
import jax
import jax.numpy as jnp
from jax.experimental import pallas as pl

_N = 10000
_C = 128
_BLK = 5000


def _body(x_ref, wcat_ref, bcat_ref, wlin_ref, blin_ref, o_ref):
    t = jnp.dot(x_ref[...], wcat_ref[...], preferred_element_type=jnp.float32)
    t = t + bcat_ref[...]
    a = t[:, :_C]
    b = t[:, _C:]
    hpre = jax.nn.sigmoid(-a) * jnp.tanh(b)
    h = jnp.where(hpre > 0, hpre, jnp.exp(hpre) - 1.0)
    o_ref[...] = (
        jnp.dot(h, wlin_ref[...], preferred_element_type=jnp.float32)
        + blin_ref[...]
    )


def kernel(x, edge_index, edge_weight, W_xz, b_xz, W_hz, b_hz, W_xr, b_xr,
           W_hr, b_hr, W_xh, b_xh, W_hh, b_hh, W_lin, b_lin):
    wcat = jnp.concatenate([W_xz, W_xh], axis=1)
    bcat = jnp.concatenate([b_xz + b_hz, b_xh + b_hh]).reshape(1, 2 * _C)
    blin = b_lin.reshape(1, _C)

    grid = (_N // _BLK,)
    return pl.pallas_call(
        _body,
        grid=grid,
        in_specs=[
            pl.BlockSpec((_BLK, _C), lambda i: (i, 0)),
            pl.BlockSpec((_C, 2 * _C), lambda i: (0, 0)),
            pl.BlockSpec((1, 2 * _C), lambda i: (0, 0)),
            pl.BlockSpec((_C, _C), lambda i: (0, 0)),
            pl.BlockSpec((1, _C), lambda i: (0, 0)),
        ],
        out_specs=pl.BlockSpec((_BLK, _C), lambda i: (i, 0)),
        out_shape=jax.ShapeDtypeStruct((_N, _C), jnp.float32),
    )(x, wcat, bcat, W_lin, blin)
